# baseline (speedup 1.0000x reference)
import jax
import jax.numpy as jnp
from jax import lax
from jax.experimental import pallas as pl
from jax.experimental.pallas import tpu as pltpu

B = 1024
K = 8192
D = 256
KT = 1024
NK = K // KT
NJ = NK // 2 + 1


def _vpu_phase(s, t, valid, cb_ref, best_val, best_idx, out_acc):
    m = jnp.max(s, axis=1, keepdims=True)
    cols = lax.broadcasted_iota(jnp.int32, s.shape, 1)
    local = jnp.min(jnp.where(s == m, cols, jnp.int32(K)), axis=1,
                    keepdims=True)
    prev = best_val[...]
    better = jnp.logical_and(m > prev, valid)
    best_val[...] = jnp.where(better, m, prev)
    best_idx[...] = jnp.where(better, local + t * KT, best_idx[...])
    oh = jnp.where(cols == local, jnp.float32(1),
                   jnp.float32(0)).astype(jnp.bfloat16)
    cand = lax.dot_general(
        oh, cb_ref[...], (((1,), (0,)), ((), ())),
        preferred_element_type=jnp.float32)
    out_acc[...] = jnp.where(better, cand, out_acc[...])


def _body(dp_ref, cba_ref, cbb_ref, cbc_ref, out_ref, idx_ref,
          best_val, best_idx, out_acc, s_a, s_b):
    j = pl.program_id(0)

    @pl.when(j == 0)
    def _init():
        best_val[...] = jnp.full(best_val.shape, -jnp.inf, best_val.dtype)
        best_idx[...] = jnp.zeros(best_idx.shape, best_idx.dtype)

    dp = dp_ref[...]
    s_a[...] = lax.dot_general(
        dp, cba_ref[...], (((1,), (1,)), ((), ())),
        preferred_element_type=jnp.float32)
    _vpu_phase(s_b[...], 2 * j - 1, j > 0, cbc_ref,
               best_val, best_idx, out_acc)
    s_b[...] = lax.dot_general(
        dp, cbb_ref[...], (((1,), (1,)), ((), ())),
        preferred_element_type=jnp.float32)
    _vpu_phase(s_a[...], 2 * j, 2 * j <= NK - 1, cba_ref,
               best_val, best_idx, out_acc)

    @pl.when(j == NJ - 1)
    def _fin():
        out_ref[...] = out_acc[...]
        idx_ref[...] = best_idx[...].reshape(8, 128)


def _assign_gather(dp_b, c_b):
    last = NK - 1
    return pl.pallas_call(
        _body,
        grid=(NJ,),
        in_specs=[
            pl.BlockSpec((B, D), lambda j: (0, 0)),
            pl.BlockSpec((KT, D), lambda j: (jnp.minimum(2 * j, last), 0)),
            pl.BlockSpec((KT, D),
                         lambda j: (jnp.minimum(2 * j + 1, last), 0)),
            pl.BlockSpec((KT, D),
                         lambda j: (jnp.maximum(2 * j - 1, 0), 0)),
        ],
        out_specs=[
            pl.BlockSpec((B, D), lambda j: (0, 0)),
            pl.BlockSpec((8, 128), lambda j: (0, 0)),
        ],
        out_shape=[
            jax.ShapeDtypeStruct((B, D), jnp.float32),
            jax.ShapeDtypeStruct((8, 128), jnp.int32),
        ],
        scratch_shapes=[
            pltpu.VMEM((B, 1), jnp.float32),
            pltpu.VMEM((B, 1), jnp.int32),
            pltpu.VMEM((B, D), jnp.float32),
            pltpu.VMEM((B, KT), jnp.float32),
            pltpu.VMEM((B, KT), jnp.float32),
        ],
    )(dp_b, c_b, c_b, c_b)


def kernel(datapoints, input_ids, batch_cos_sim, centroid):
    dp = jax.lax.stop_gradient(datapoints)
    ndp = jnp.maximum(jnp.sqrt(jnp.sum(dp * dp, axis=-1, keepdims=True)),
                      1e-8)
    dp_b = (dp / ndp).astype(jnp.bfloat16)
    mx = jnp.maximum(
        jnp.sqrt(jnp.sum(centroid * centroid, axis=-1, keepdims=True)), 1e-8)
    c_b = (centroid / mx).astype(jnp.bfloat16)
    rows, idx = _assign_gather(dp_b, c_b)
    return rows * jnp.take(mx.reshape(K), idx.reshape(B))[:, None]


# speedup vs baseline: 1.2880x; 1.2880x over previous
import jax
import jax.numpy as jnp
from jax import lax
from jax.experimental import pallas as pl
from jax.experimental.pallas import tpu as pltpu

B = 1024
K = 8192
D = 256
KT = 1024
NK = K // KT
NJ = NK // 2 + 1


def _vpu_phase(s, t, valid, cr_ref, best_val, best_idx, out_acc):
    m = jnp.max(s, axis=1, keepdims=True)
    cols = lax.broadcasted_iota(jnp.int32, s.shape, 1)
    local = jnp.min(jnp.where(s == m, cols, jnp.int32(K)), axis=1,
                    keepdims=True)
    prev = best_val[...]
    better = jnp.logical_and(m > prev, valid)
    best_val[...] = jnp.where(better, m, prev)
    best_idx[...] = jnp.where(better, local + t * KT, best_idx[...])
    oh = jnp.where(cols == local, jnp.float32(1),
                   jnp.float32(0)).astype(jnp.bfloat16)
    cand = lax.dot_general(
        oh, cr_ref[...], (((1,), (0,)), ((), ())),
        preferred_element_type=jnp.float32)
    out_acc[...] = jnp.where(better, cand, out_acc[...])


def _body(dp_ref, cba_ref, cbb_ref, crb_ref, cra_ref, out_ref,
          best_val, best_idx, out_acc, s_a, s_b):
    j = pl.program_id(0)

    @pl.when(j == 0)
    def _init():
        best_val[...] = jnp.full(best_val.shape, -jnp.inf, best_val.dtype)
        best_idx[...] = jnp.zeros(best_idx.shape, best_idx.dtype)

    dp = dp_ref[...]
    s_a[...] = lax.dot_general(
        dp, cba_ref[...], (((1,), (1,)), ((), ())),
        preferred_element_type=jnp.float32)
    _vpu_phase(s_b[...], 2 * j - 1, j > 0, crb_ref,
               best_val, best_idx, out_acc)
    s_b[...] = lax.dot_general(
        dp, cbb_ref[...], (((1,), (1,)), ((), ())),
        preferred_element_type=jnp.float32)
    _vpu_phase(s_a[...], 2 * j, 2 * j <= NK - 1, cra_ref,
               best_val, best_idx, out_acc)

    @pl.when(j == NJ - 1)
    def _fin():
        out_ref[...] = out_acc[...]


def _assign_gather(dp_b, c_b, c_r):
    last = NK - 1
    return pl.pallas_call(
        _body,
        grid=(NJ,),
        in_specs=[
            pl.BlockSpec((B, D), lambda j: (0, 0)),
            pl.BlockSpec((KT, D), lambda j: (jnp.minimum(2 * j, last), 0)),
            pl.BlockSpec((KT, D),
                         lambda j: (jnp.minimum(2 * j + 1, last), 0)),
            pl.BlockSpec((KT, D),
                         lambda j: (jnp.maximum(2 * j - 1, 0), 0)),
            pl.BlockSpec((KT, D), lambda j: (jnp.minimum(2 * j, last), 0)),
        ],
        out_specs=pl.BlockSpec((B, D), lambda j: (0, 0)),
        out_shape=jax.ShapeDtypeStruct((B, D), jnp.float32),
        scratch_shapes=[
            pltpu.VMEM((B, 1), jnp.float32),
            pltpu.VMEM((B, 1), jnp.int32),
            pltpu.VMEM((B, D), jnp.float32),
            pltpu.VMEM((B, KT), jnp.float32),
            pltpu.VMEM((B, KT), jnp.float32),
        ],
    )(dp_b, c_b, c_b, c_r, c_r)


def kernel(datapoints, input_ids, batch_cos_sim, centroid):
    dp = jax.lax.stop_gradient(datapoints)
    ndp = jnp.maximum(jnp.sqrt(jnp.sum(dp * dp, axis=-1, keepdims=True)),
                      1e-8)
    dp_b = (dp / ndp).astype(jnp.bfloat16)
    mx = jnp.maximum(
        jnp.sqrt(jnp.sum(centroid * centroid, axis=-1, keepdims=True)), 1e-8)
    c_b = (centroid / mx).astype(jnp.bfloat16)
    c_r = centroid.astype(jnp.bfloat16)
    return _assign_gather(dp_b, c_b, c_r)
